# baseline (speedup 1.0000x reference)
import functools

import jax
import jax.numpy as jnp
from jax import lax
from jax.experimental import pallas as pl
from jax.experimental.pallas import tpu as pltpu
from jax.experimental.pallas import tpu_sc as plsc

VOCAB = 1000000
EMBED_DIM = 32
BATCH = 4096
HIST = 50

_info = plsc.get_sparse_core_info()
_NC, _NS = _info.num_cores, _info.num_subcores
_NW = _NC * _NS
_TOTAL = BATCH * HIST
_PER_W = _TOTAL // _NW
_K = 128
_C = _PER_W // _K

_mesh = plsc.VectorSubcoreMesh(core_axis_name="c", subcore_axis_name="s")


@functools.partial(
    pl.kernel,
    out_type=jax.ShapeDtypeStruct((_TOTAL, EMBED_DIM), jnp.float32),
    mesh=_mesh,
    compiler_params=pltpu.CompilerParams(use_tc_tiling_on_sc=False),
    scratch_types=[
        pltpu.VMEM((_C, _K), jnp.int32),
        pltpu.VMEM((_K, EMBED_DIM), jnp.float32),
        pltpu.SemaphoreType.DMA,
    ],
)
def _gather_kernel(idx_hbm, table_hbm, out_hbm, idx_v, rows_v, sem):
    wid = lax.axis_index("s") * _NC + lax.axis_index("c")
    base = wid * _PER_W
    pltpu.sync_copy(idx_hbm.at[wid], idx_v)

    def body(c, _):
        pltpu.async_copy(table_hbm.at[idx_v.at[c]], rows_v, sem).wait()
        pltpu.sync_copy(rows_v, out_hbm.at[pl.ds(base + c * _K, _K)])
        return 0

    lax.fori_loop(0, _C, body, 0)


def kernel(indices, table):
    idx3 = indices.reshape(_NW, _C, _K).astype(jnp.int32)
    out = _gather_kernel(idx3, table)
    return out.reshape(BATCH, HIST, EMBED_DIM)


# speedup vs baseline: 1.0399x; 1.0399x over previous
import functools

import jax
import jax.numpy as jnp
from jax import lax
from jax.experimental import pallas as pl
from jax.experimental.pallas import tpu as pltpu
from jax.experimental.pallas import tpu_sc as plsc

VOCAB = 1000000
EMBED_DIM = 32
BATCH = 4096
HIST = 50

_info = plsc.get_sparse_core_info()
_NC, _NS = _info.num_cores, _info.num_subcores
_NW = _NC * _NS
_TOTAL = BATCH * HIST
_PER_W = _TOTAL // _NW
_K = 128
_KCH = 5
_B = _K * _KCH
_NB = _PER_W // _B
_C = _PER_W // _K

_mesh = plsc.VectorSubcoreMesh(core_axis_name="c", subcore_axis_name="s")


@functools.partial(
    pl.kernel,
    out_type=jax.ShapeDtypeStruct((_TOTAL, EMBED_DIM), jnp.float32),
    mesh=_mesh,
    compiler_params=pltpu.CompilerParams(use_tc_tiling_on_sc=False),
    scratch_types=[
        pltpu.VMEM((_C, _K), jnp.int32),
        pltpu.VMEM((_B, EMBED_DIM), jnp.float32),
        pltpu.VMEM((_B, EMBED_DIM), jnp.float32),
        pltpu.SemaphoreType.DMA,
        pltpu.SemaphoreType.DMA,
        pltpu.SemaphoreType.DMA,
        pltpu.SemaphoreType.DMA,
    ],
)
def _gather_kernel(idx_hbm, table_hbm, out_hbm, idx_v, rows0, rows1,
                   gsem0, gsem1, osem0, osem1):
    wid = lax.axis_index("s") * _NC + lax.axis_index("c")
    base = wid * _PER_W
    pltpu.sync_copy(idx_hbm.at[wid], idx_v)

    def fire_g(t, rows, gsem):
        for j in range(_KCH):
            pltpu.make_async_copy(
                table_hbm.at[idx_v.at[t * _KCH + j]],
                rows.at[pl.ds(j * _K, _K)], gsem,
            ).start()

    def drain_g(rows, gsem):
        for j in range(_KCH):
            pltpu.make_async_copy(
                table_hbm.at[pl.ds(0, _K)], rows.at[pl.ds(j * _K, _K)], gsem
            ).wait()

    def fire_o(t, rows, osem):
        pltpu.make_async_copy(
            rows, out_hbm.at[pl.ds(base + t * _B, _B)], osem,
        ).start()

    def drain_o(rows, osem):
        pltpu.make_async_copy(
            rows, out_hbm.at[pl.ds(base, _B)], osem
        ).wait()

    def handle(t, rows, gsem, osem, rows_n, gsem_n, osem_n, fire_next):
        drain_g(rows, gsem)
        fire_o(t, rows, osem)
        if fire_next:
            drain_o(rows_n, osem_n)
            fire_g(t + 1, rows_n, gsem_n)

    fire_g(0, rows0, gsem0)
    drain_g(rows0, gsem0)
    fire_o(0, rows0, osem0)
    fire_g(1, rows1, gsem1)

    def pair(i, _):
        t = 2 * i + 1
        handle(t, rows1, gsem1, osem1, rows0, gsem0, osem0, True)
        handle(t + 1, rows0, gsem0, osem0, rows1, gsem1, osem1, True)
        return 0

    lax.fori_loop(0, (_NB - 4) // 2, pair, 0)

    handle(_NB - 3, rows1, gsem1, osem1, rows0, gsem0, osem0, True)
    handle(_NB - 2, rows0, gsem0, osem0, rows1, gsem1, osem1, True)
    handle(_NB - 1, rows1, gsem1, osem1, None, None, None, False)
    drain_o(rows0, osem0)
    drain_o(rows1, osem1)


def kernel(indices, table):
    idx3 = indices.reshape(_NW, _C, _K).astype(jnp.int32)
    out = _gather_kernel(idx3, table)
    return out.reshape(BATCH, HIST, EMBED_DIM)
